# baseline (device time: 38421 ns/iter reference)
import jax
import jax.numpy as jnp
from jax import lax
from jax.experimental import pallas as pl
from jax.experimental.pallas import tpu as pltpu

N_LAYERS = 3
MASKS = (1, 3, 4)
CH = 8
N_EX = N_LAYERS * len(MASKS) * CH

_sem_signal = getattr(pltpu, "semaphore_signal", None) or getattr(pl, "semaphore_signal")
_sem_wait = getattr(pltpu, "semaphore_wait", None) or getattr(pl, "semaphore_wait")
_DeviceIdType = getattr(pl, "DeviceIdType", None) or getattr(pltpu, "DeviceIdType")
_CompilerParams = getattr(pltpu, "CompilerParams", None) or getattr(
    pltpu, "TPUCompilerParams"
)
_ANY = getattr(pltpu, "ANY", None) or getattr(pl, "ANY", None)
if _ANY is None:
    _ANY = pltpu.MemorySpace.HBM


def kernel(x, Win0, Wout0, Win1, Wout1, Win2, Wout2):
    b, d = x.shape
    dh = Win0.shape[1]
    cw = d // CH

    def body(
        x_ref,
        win0_hbm,
        wout0_hbm,
        win1_hbm,
        wout1_hbm,
        win2_hbm,
        wout2_hbm,
        out_ref,
        w0,
        o0,
        w1,
        o1,
        w2,
        o2,
        sb,
        rv,
        send_sems,
        recv_sems,
        cp_sems,
    ):
        my = lax.axis_index("i")
        bf16 = jnp.bfloat16

        srcs = (win0_hbm, wout0_hbm, win1_hbm, wout1_hbm, win2_hbm, wout2_hbm)
        dsts = (w0, o0, w1, o1, w2, o2)
        cps = []
        for i in range(6):
            c = pltpu.make_async_copy(srcs[i], dsts[i], cp_sems.at[i])
            c.start()
            cps.append(c)
        wbufs = (w0, w1, w2)
        obufs = (o0, o1, o2)

        barrier = pltpu.get_barrier_semaphore()
        for mask in MASKS:
            _sem_signal(
                barrier,
                inc=1,
                device_id=(my ^ mask,),
                device_id_type=_DeviceIdType.MESH,
            )
        _sem_wait(barrier, len(MASKS))

        def make_rdma(l, s, c):
            idx = (l * len(MASKS) + s) * CH + c
            return idx, pltpu.make_async_remote_copy(
                src_ref=sb.at[idx],
                dst_ref=rv.at[idx],
                send_sem=send_sems.at[idx],
                recv_sem=recv_sems.at[idx],
                device_id=(my ^ MASKS[s],),
                device_id_type=_DeviceIdType.MESH,
            )

        rdmas = {}

        cps[0].wait()
        h = jnp.maximum(
            jnp.dot(
                x_ref[:, :].astype(bf16),
                w0[:, :].astype(bf16),
                preferred_element_type=jnp.float32,
            ),
            0.0,
        )
        for l in range(N_LAYERS):
            cps[2 * l + 1].wait()
            hb = h.astype(bf16)
            woutb = obufs[l][:, :].astype(bf16)
            ps = []
            for c in range(CH):
                p = jnp.dot(
                    hb,
                    woutb[:, c * cw : (c + 1) * cw],
                    preferred_element_type=jnp.float32,
                )
                ps.append(p)
                idx, rdma = make_rdma(l, 0, c)
                sb[idx, :, :] = p.astype(bf16)
                rdmas[idx] = rdma
                rdma.start()

            if l + 1 < N_LAYERS:
                cps[2 * (l + 1)].wait()
                winb_next = wbufs[l + 1][:, :].astype(bf16)
            else:
                winb_next = None

            hn = None
            for s in range(len(MASKS)):
                for c in range(CH):
                    idx = (l * len(MASKS) + s) * CH + c
                    rdmas[idx].wait_recv()
                    ps[c] = ps[c] + rv[idx, :, :].astype(jnp.float32)
                    if s + 1 < len(MASKS):
                        idx2, rdma2 = make_rdma(l, s + 1, c)
                        sb[idx2, :, :] = ps[c].astype(bf16)
                        rdmas[idx2] = rdma2
                        rdma2.start()
                    elif l + 1 < N_LAYERS:
                        contrib = jnp.dot(
                            ps[c].astype(bf16),
                            winb_next[c * cw : (c + 1) * cw, :],
                            preferred_element_type=jnp.float32,
                        )
                        hn = contrib if hn is None else hn + contrib
                    else:
                        out_ref[:, c * cw : (c + 1) * cw] = ps[c]
            if l + 1 < N_LAYERS:
                h = jnp.maximum(hn, 0.0)

        for idx, rdma in rdmas.items():
            rdma.wait_send()

    weight_spec = pl.BlockSpec(memory_space=_ANY)
    return pl.pallas_call(
        body,
        out_shape=jax.ShapeDtypeStruct((b, d), jnp.float32),
        in_specs=[pl.BlockSpec(memory_space=pltpu.VMEM)] + [weight_spec] * 6,
        out_specs=pl.BlockSpec(memory_space=pltpu.VMEM),
        scratch_shapes=[
            pltpu.VMEM((d, dh), jnp.float32),
            pltpu.VMEM((dh, d), jnp.float32),
            pltpu.VMEM((d, dh), jnp.float32),
            pltpu.VMEM((dh, d), jnp.float32),
            pltpu.VMEM((d, dh), jnp.float32),
            pltpu.VMEM((dh, d), jnp.float32),
            pltpu.VMEM((N_EX, b, cw), jnp.bfloat16),
            pltpu.VMEM((N_EX, b, cw), jnp.bfloat16),
            pltpu.SemaphoreType.DMA((N_EX,)),
            pltpu.SemaphoreType.DMA((N_EX,)),
            pltpu.SemaphoreType.DMA((6,)),
        ],
        compiler_params=_CompilerParams(
            collective_id=0, vmem_limit_bytes=100 * 1024 * 1024
        ),
    )(x, Win0, Wout0, Win1, Wout1, Win2, Wout2)


# device time: 37943 ns/iter; 1.0126x vs baseline; 1.0126x over previous
import jax
import jax.numpy as jnp
from jax import lax
from jax.experimental import pallas as pl
from jax.experimental.pallas import tpu as pltpu

N_LAYERS = 3
MASKS = (1, 3, 4)
CH = 4
N_EX = N_LAYERS * len(MASKS) * CH

_sem_signal = getattr(pltpu, "semaphore_signal", None) or getattr(pl, "semaphore_signal")
_sem_wait = getattr(pltpu, "semaphore_wait", None) or getattr(pl, "semaphore_wait")
_DeviceIdType = getattr(pl, "DeviceIdType", None) or getattr(pltpu, "DeviceIdType")
_CompilerParams = getattr(pltpu, "CompilerParams", None) or getattr(
    pltpu, "TPUCompilerParams"
)
_ANY = getattr(pltpu, "ANY", None) or getattr(pl, "ANY", None)
if _ANY is None:
    _ANY = pltpu.MemorySpace.HBM


def kernel(x, Win0, Wout0, Win1, Wout1, Win2, Wout2):
    b, d = x.shape
    dh = Win0.shape[1]
    cw = d // CH

    def body(
        x_ref,
        win0_hbm,
        wout0_hbm,
        win1_hbm,
        wout1_hbm,
        win2_hbm,
        wout2_hbm,
        out_ref,
        w0,
        o0,
        w1,
        o1,
        w2,
        o2,
        sb,
        rv,
        send_sems,
        recv_sems,
        cp_sems,
    ):
        my = lax.axis_index("i")
        bf16 = jnp.bfloat16

        srcs = (win0_hbm, wout0_hbm, win1_hbm, wout1_hbm, win2_hbm, wout2_hbm)
        dsts = (w0, o0, w1, o1, w2, o2)
        cps = []
        for i in range(6):
            c = pltpu.make_async_copy(srcs[i], dsts[i], cp_sems.at[i])
            c.start()
            cps.append(c)
        wbufs = (w0, w1, w2)
        obufs = (o0, o1, o2)

        barrier = pltpu.get_barrier_semaphore()
        for mask in MASKS:
            _sem_signal(
                barrier,
                inc=1,
                device_id=(my ^ mask,),
                device_id_type=_DeviceIdType.MESH,
            )
        _sem_wait(barrier, len(MASKS))

        def make_rdma(l, s, c):
            idx = (l * len(MASKS) + s) * CH + c
            return idx, pltpu.make_async_remote_copy(
                src_ref=sb.at[idx],
                dst_ref=rv.at[idx],
                send_sem=send_sems.at[idx],
                recv_sem=recv_sems.at[idx],
                device_id=(my ^ MASKS[s],),
                device_id_type=_DeviceIdType.MESH,
            )

        rdmas = {}

        cps[0].wait()
        h = jnp.maximum(
            jnp.dot(
                x_ref[:, :].astype(bf16),
                w0[:, :].astype(bf16),
                preferred_element_type=jnp.float32,
            ),
            0.0,
        )
        for l in range(N_LAYERS):
            cps[2 * l + 1].wait()
            hb = h.astype(bf16)
            woutb = obufs[l][:, :].astype(bf16)
            ps = []
            for c in range(CH):
                p = jnp.dot(
                    hb,
                    woutb[:, c * cw : (c + 1) * cw],
                    preferred_element_type=jnp.float32,
                )
                ps.append(p)
                idx, rdma = make_rdma(l, 0, c)
                sb[idx, :, :] = p.astype(bf16)
                rdmas[idx] = rdma
                rdma.start()

            if l + 1 < N_LAYERS:
                cps[2 * (l + 1)].wait()
                winb_next = wbufs[l + 1][:, :].astype(bf16)
            else:
                winb_next = None

            hn = None
            for s in range(len(MASKS)):
                for c in range(CH):
                    idx = (l * len(MASKS) + s) * CH + c
                    rdmas[idx].wait_recv()
                    ps[c] = ps[c] + rv[idx, :, :].astype(jnp.float32)
                    if s + 1 < len(MASKS):
                        idx2, rdma2 = make_rdma(l, s + 1, c)
                        sb[idx2, :, :] = ps[c].astype(bf16)
                        rdmas[idx2] = rdma2
                        rdma2.start()
                    elif l + 1 < N_LAYERS:
                        contrib = jnp.dot(
                            ps[c].astype(bf16),
                            winb_next[c * cw : (c + 1) * cw, :],
                            preferred_element_type=jnp.float32,
                        )
                        hn = contrib if hn is None else hn + contrib
                    else:
                        out_ref[:, c * cw : (c + 1) * cw] = ps[c]
            if l + 1 < N_LAYERS:
                h = jnp.maximum(hn, 0.0)

        for idx, rdma in rdmas.items():
            rdma.wait_send()

    weight_spec = pl.BlockSpec(memory_space=_ANY)
    return pl.pallas_call(
        body,
        out_shape=jax.ShapeDtypeStruct((b, d), jnp.float32),
        in_specs=[pl.BlockSpec(memory_space=pltpu.VMEM)] + [weight_spec] * 6,
        out_specs=pl.BlockSpec(memory_space=pltpu.VMEM),
        scratch_shapes=[
            pltpu.VMEM((d, dh), jnp.float32),
            pltpu.VMEM((dh, d), jnp.float32),
            pltpu.VMEM((d, dh), jnp.float32),
            pltpu.VMEM((dh, d), jnp.float32),
            pltpu.VMEM((d, dh), jnp.float32),
            pltpu.VMEM((dh, d), jnp.float32),
            pltpu.VMEM((N_EX, b, cw), jnp.bfloat16),
            pltpu.VMEM((N_EX, b, cw), jnp.bfloat16),
            pltpu.SemaphoreType.DMA((N_EX,)),
            pltpu.SemaphoreType.DMA((N_EX,)),
            pltpu.SemaphoreType.DMA((6,)),
        ],
        compiler_params=_CompilerParams(
            collective_id=0, vmem_limit_bytes=100 * 1024 * 1024
        ),
    )(x, Win0, Wout0, Win1, Wout1, Win2, Wout2)
